# Initial kernel scaffold; baseline (speedup 1.0000x reference)
import jax
import jax.numpy as jnp
from jax.experimental import pallas as pl


def kernel(inputs, mask, edge_index, W1, b1, gat_W, gat_a, gat_fc_W, gat_fc_b, gat_bn_g, gat_bn_b, avg_fc_W, avg_fc_b, avg_bn_g, avg_bn_b, W2, b2, bn2_g, bn2_b):
    raise NotImplementedError("write your pallas kernel here")


# speedup vs baseline: 1.0161x; 1.0161x over previous
import jax
import jax.numpy as jnp
from jax.experimental import pallas as pl

N_LAYERS = 15
D = 128
BLK = 1024
PN = 10240

_INTERPRET = False


def _mm_body(x_ref, w_ref, b_ref, o_ref):
    o_ref[...] = jnp.dot(x_ref[...], w_ref[...],
                         preferred_element_type=jnp.float32) + b_ref[...]


def _mm(x, w, b):
    n, k = x.shape
    m = w.shape[1]
    xp = jnp.pad(x.astype(jnp.bfloat16), ((0, PN - n), (0, 0)))
    y = pl.pallas_call(
        _mm_body,
        grid=(PN // BLK,),
        in_specs=[
            pl.BlockSpec((BLK, k), lambda i: (i, 0)),
            pl.BlockSpec((k, m), lambda i: (0, 0)),
            pl.BlockSpec((1, m), lambda i: (0, 0)),
        ],
        out_specs=pl.BlockSpec((BLK, m), lambda i: (i, 0)),
        out_shape=jax.ShapeDtypeStruct((PN, m), jnp.float32),
        interpret=_INTERPRET,
    )(xp, w.astype(jnp.bfloat16), b.reshape(1, -1))
    return y[:n]


def _bn_pre(x2, g, b, eps=1e-5):
    mu = x2.mean(axis=0)
    var = x2.var(axis=0)
    return (x2 - mu) * jax.lax.rsqrt(var + eps) * g + b


def _gc1x1(x, W, b, g=None, be=None):
    B, N, C = x.shape
    x2 = x.reshape(-1, C)
    if g is not None:
        x2 = _bn_pre(x2, g, be)
    y = _mm(x2, W, b)
    return y.reshape(B, N, W.shape[1])


def _gat(x, Wg, a, src, dst, n):
    def single(h):
        hw = _mm(h, Wg, jnp.zeros((Wg.shape[1],), jnp.float32))
        e = (hw @ a[0])[src] + (hw @ a[1])[dst]
        e = jnp.where(e > 0, e, 0.2 * e)
        m = jax.ops.segment_max(e, dst, num_segments=n)
        m = jnp.where(m < -1e30, 0.0, m)
        ex = jnp.exp(e - m[dst])
        den = jax.ops.segment_sum(ex, dst, num_segments=n)
        num = jax.ops.segment_sum(ex[:, None] * hw[src], dst, num_segments=n)
        return num / (den[:, None] + 1e-16)
    B = x.shape[0]
    return jnp.stack([single(x[b]) for b in range(B)])


def kernel(inputs, mask, edge_index, W1, b1, gat_W, gat_a, gat_fc_W, gat_fc_b,
           gat_bn_g, gat_bn_b, avg_fc_W, avg_fc_b, avg_bn_g, avg_bn_b,
           W2, b2, bn2_g, bn2_b):
    B, N, _ = inputs.shape
    src = edge_index[0]
    dst = edge_index[1]
    x = _gc1x1(inputs, W1, b1)
    gi = 0
    ai = 0
    for i in range(N_LAYERS):
        x0 = x
        if i % 2 == 0:
            for j in range(2):
                x = jax.nn.elu(x)
                agg = _gat(x, gat_W[gi, j], gat_a[gi, j], src, dst, N)
                x = _gc1x1(jnp.concatenate([x, agg], axis=2),
                           gat_fc_W[gi, j], gat_fc_b[gi, j],
                           gat_bn_g[gi, j], gat_bn_b[gi, j])
            gi += 1
        else:
            for j in range(2):
                x = jax.nn.elu(x)
                avg = (x * mask).sum(axis=1, keepdims=True) / \
                    mask.sum(axis=1, keepdims=True)
                x = _gc1x1(jnp.concatenate(
                    [x, jnp.broadcast_to(avg, x.shape)], axis=2),
                    avg_fc_W[ai, j], avg_fc_b[ai, j],
                    avg_bn_g[ai, j], avg_bn_b[ai, j])
            ai += 1
        x = x + x0
    x = jax.nn.elu(x)
    x = _gc1x1(x, W2, b2, bn2_g, bn2_b)
    return x + inputs[:, :, : x.shape[2]]
